# baseline (speedup 1.0000x reference)
import jax
import jax.numpy as jnp
from jax import lax
from jax.experimental import pallas as pl
from jax.experimental.pallas import tpu as pltpu
from jax.experimental.pallas import tpu_sc as plsc

VOCAB = 10000
EMB = 256
B = 4096
L = 200

_INFO = plsc.get_sparse_core_info()
NC = _INFO.num_cores
NS = _INFO.num_subcores
LANES = _INFO.num_lanes
NW = NC * NS
ROWS_PER_W = B // NW
G_PER_W = ROWS_PER_W // LANES

_PROJ_CHUNK = 2000
_PROJ_NCHUNK = VOCAB // _PROJ_CHUNK


def _proj_body(emb_hbm, w_ref, out_ref, *scratch):
    bufs = scratch[:_PROJ_NCHUNK]
    sems = scratch[_PROJ_NCHUNK:]
    w = w_ref[0] * (1.0 / L)
    cps = [
        pltpu.async_copy(
            emb_hbm.at[pl.ds(i * _PROJ_CHUNK, _PROJ_CHUNK)], bufs[i], sems[i]
        )
        for i in range(_PROJ_NCHUNK)
    ]
    for i in range(_PROJ_NCHUNK):
        cps[i].wait()
        out_ref[pl.ds(i * _PROJ_CHUNK, _PROJ_CHUNK)] = jax.lax.dot_general(
            bufs[i][...], w, (((1,), (0,)), ((), ())),
            preferred_element_type=jnp.float32,
        )


_proj_call = pl.pallas_call(
    _proj_body,
    in_specs=[
        pl.BlockSpec(memory_space=pltpu.HBM),
        pl.BlockSpec((1, EMB), lambda: (0, 0)),
    ],
    out_specs=pl.BlockSpec((VOCAB,), lambda: (0,)),
    out_shape=jax.ShapeDtypeStruct((VOCAB,), jnp.float32),
    scratch_shapes=(
        [pltpu.VMEM((_PROJ_CHUNK, EMB), jnp.float32)] * _PROJ_NCHUNK
        + [pltpu.SemaphoreType.DMA] * _PROJ_NCHUNK
    ),
)


def _sc_body(proj_hbm, resp_hbm, b_hbm, out_hbm, proj_v, resp_v, b_v, out_v, sem):
    wid = lax.axis_index("s") * NC + lax.axis_index("c")
    cp1 = pltpu.async_copy(proj_hbm, proj_v, sem)
    cp2 = pltpu.async_copy(
        resp_hbm.at[:, pl.ds(wid * ROWS_PER_W, ROWS_PER_W)], resp_v, sem
    )
    cp3 = pltpu.async_copy(b_hbm, b_v, sem)
    cp1.wait()
    cp2.wait()
    cp3.wait()
    bvec = b_v[...]

    def body(l, accs):
        new = []
        for g in range(G_PER_W):
            tok = resp_v[l, pl.ds(g * LANES, LANES)]
            new.append(accs[g] + plsc.load_gather(proj_v, [tok]))
        return tuple(new)

    accs = lax.fori_loop(
        0, L, body, tuple(jnp.zeros((LANES,), jnp.float32) for _ in range(G_PER_W))
    )
    for g in range(G_PER_W):
        out_v[pl.ds(g * LANES, LANES)] = accs[g] + bvec
    pltpu.sync_copy(out_v, out_hbm.at[pl.ds(wid * ROWS_PER_W, ROWS_PER_W)])


_sc_call = pl.kernel(
    _sc_body,
    out_type=jax.ShapeDtypeStruct((B,), jnp.float32),
    mesh=plsc.VectorSubcoreMesh(core_axis_name="c", subcore_axis_name="s"),
    compiler_params=pltpu.CompilerParams(needs_layout_passes=False),
    scratch_types=[
        pltpu.VMEM((VOCAB,), jnp.float32),
        pltpu.VMEM((L, ROWS_PER_W), jnp.int32),
        pltpu.VMEM((LANES,), jnp.float32),
        pltpu.VMEM((ROWS_PER_W,), jnp.float32),
        pltpu.SemaphoreType.DMA,
    ],
)


@jax.jit
def kernel(response, emb_table, W, b):
    proj = _proj_call(emb_table, W)
    b16 = jnp.broadcast_to(b, (LANES,)).astype(jnp.float32)
    out = _sc_call(proj, response.T, b16)
    return out.reshape(B, 1)


# speedup vs baseline: 1.0238x; 1.0238x over previous
import jax
import jax.numpy as jnp
from jax import lax
from jax.experimental import pallas as pl
from jax.experimental.pallas import tpu as pltpu
from jax.experimental.pallas import tpu_sc as plsc

VOCAB = 10000
EMB = 256
B = 4096
L = 200

_INFO = plsc.get_sparse_core_info()
NC = _INFO.num_cores
NS = _INFO.num_subcores
LANES = _INFO.num_lanes
NW = NC * NS
ROWS_PER_W = B // NW
G_PER_W = ROWS_PER_W // LANES

_PROJ_CHUNK = 2000
_PROJ_NCHUNK = VOCAB // _PROJ_CHUNK


def _proj_body(emb_hbm, w_ref, b_ref, out_ref, b16_ref, *scratch):
    bufs = scratch[:_PROJ_NCHUNK]
    sems = scratch[_PROJ_NCHUNK:]
    b16_ref[:] = jnp.broadcast_to(b_ref[0], (LANES,))
    w = w_ref[0] * (1.0 / L)
    cps = [
        pltpu.async_copy(
            emb_hbm.at[pl.ds(i * _PROJ_CHUNK, _PROJ_CHUNK)], bufs[i], sems[i]
        )
        for i in range(_PROJ_NCHUNK)
    ]
    for i in range(_PROJ_NCHUNK):
        cps[i].wait()
        out_ref[pl.ds(i * _PROJ_CHUNK, _PROJ_CHUNK)] = jax.lax.dot_general(
            bufs[i][...], w, (((1,), (0,)), ((), ())),
            preferred_element_type=jnp.float32,
        )


_proj_call = pl.pallas_call(
    _proj_body,
    in_specs=[
        pl.BlockSpec(memory_space=pltpu.HBM),
        pl.BlockSpec((1, EMB), lambda: (0, 0)),
        pl.BlockSpec((1,), lambda: (0,)),
    ],
    out_specs=[
        pl.BlockSpec((VOCAB,), lambda: (0,)),
        pl.BlockSpec((LANES,), lambda: (0,)),
    ],
    out_shape=[
        jax.ShapeDtypeStruct((VOCAB,), jnp.float32),
        jax.ShapeDtypeStruct((LANES,), jnp.float32),
    ],
    scratch_shapes=(
        [pltpu.VMEM((_PROJ_CHUNK, EMB), jnp.float32)] * _PROJ_NCHUNK
        + [pltpu.SemaphoreType.DMA] * _PROJ_NCHUNK
    ),
)


def _sc_body(proj_hbm, resp_hbm, b_hbm, out_hbm, proj_v, resp_v, b_v, out_v, sem):
    wid = lax.axis_index("s") * NC + lax.axis_index("c")
    cp1 = pltpu.async_copy(proj_hbm, proj_v, sem)
    cp2 = pltpu.async_copy(
        resp_hbm.at[:, pl.ds(wid * ROWS_PER_W, ROWS_PER_W)], resp_v, sem
    )
    cp3 = pltpu.async_copy(b_hbm, b_v, sem)
    cp1.wait()
    cp2.wait()
    cp3.wait()
    bvec = b_v[...]

    def step(l, accs):
        new = []
        for g in range(G_PER_W):
            tok = resp_v[l, pl.ds(g * LANES, LANES)]
            new.append(accs[g] + plsc.load_gather(proj_v, [tok]))
        return tuple(new)

    def body(i, accs):
        return step(2 * i + 1, step(2 * i, accs))

    accs = lax.fori_loop(
        0, L // 2, body,
        tuple(jnp.zeros((LANES,), jnp.float32) for _ in range(G_PER_W)),
    )
    for g in range(G_PER_W):
        out_v[pl.ds(g * LANES, LANES)] = accs[g] + bvec
    pltpu.sync_copy(out_v, out_hbm.at[pl.ds(wid * ROWS_PER_W, ROWS_PER_W)])


_sc_call = pl.kernel(
    _sc_body,
    out_type=jax.ShapeDtypeStruct((B,), jnp.float32),
    mesh=plsc.VectorSubcoreMesh(core_axis_name="c", subcore_axis_name="s"),
    compiler_params=pltpu.CompilerParams(needs_layout_passes=False),
    scratch_types=[
        pltpu.VMEM((VOCAB,), jnp.float32),
        pltpu.VMEM((L, ROWS_PER_W), jnp.int32),
        pltpu.VMEM((LANES,), jnp.float32),
        pltpu.VMEM((ROWS_PER_W,), jnp.float32),
        pltpu.SemaphoreType.DMA,
    ],
)


@jax.jit
def kernel(response, emb_table, W, b):
    proj, b16 = _proj_call(emb_table, W, b)
    out = _sc_call(proj, response.T, b16)
    return out.reshape(B, 1)
